# Initial kernel scaffold; baseline (speedup 1.0000x reference)
import jax
import jax.numpy as jnp
from jax.experimental import pallas as pl


def kernel(x, edge_index, W1, att_src1, att_dst1, b1, W2, att_src2, att_dst2, b2, W_mu, b_mu, W_ls, b_ls):
    raise NotImplementedError("write your pallas kernel here")


# speedup vs baseline: 25.8860x; 25.8860x over previous
import functools

import jax
import jax.numpy as jnp
from jax import lax
from jax.experimental import pallas as pl
from jax.experimental.pallas import tpu as pltpu
from jax.experimental.pallas import tpu_sc as plsc

N = 10000
E = 320000
IN_CH = 128
HID = 64
OUT_CH = 32
HEADS = 4

_BN = 400
_GRID = N // _BN

_B = 80
_NT = 16
_NPAD = 10240
_RPT = _NPAD // _NT
_EPT_A = E // _NT
_EPT_BC = E // (2 * _NT)

_MESH = plsc.VectorSubcoreMesh(core_axis_name="c", subcore_axis_name="s")
_VSH = pltpu.MemorySpace.VMEM_SHARED
_SC_PARAMS = pltpu.CompilerParams(needs_layout_passes=False)


def _lrelu_exp(a):
    return jnp.exp(jnp.maximum(a, 0.2 * a))


def _elu(z):
    return jnp.where(z > 0, z, jnp.exp(z) - 1.0)


def _tc1_body(x_ref, w1_ref, as_ref, ad_ref, xph_ref, a1s_ref, a1d_ref):
    xp = jnp.dot(x_ref[...], w1_ref[...], preferred_element_type=jnp.float32)
    for h in range(HEADS):
        seg = xp[:, h * 64:(h + 1) * 64]
        xph_ref[h] = jnp.concatenate([seg, jnp.zeros_like(seg)], axis=1)
        a1s_ref[h, :, 0] = jnp.sum(seg * as_ref[h][None, :], axis=1)
        a1d_ref[h, :, 0] = jnp.sum(seg * ad_ref[h][None, :], axis=1)


def _tc1(x, W1, att_src1, att_dst1):
    return pl.pallas_call(
        _tc1_body,
        grid=(_GRID,),
        in_specs=[
            pl.BlockSpec((_BN, IN_CH), lambda i: (i, 0)),
            pl.BlockSpec((IN_CH, 256), lambda i: (0, 0)),
            pl.BlockSpec((HEADS, HID), lambda i: (0, 0)),
            pl.BlockSpec((HEADS, HID), lambda i: (0, 0)),
        ],
        out_specs=[
            pl.BlockSpec((HEADS, _BN, 128), lambda i: (0, i, 0)),
            pl.BlockSpec((HEADS, _BN, 1), lambda i: (0, i, 0)),
            pl.BlockSpec((HEADS, _BN, 1), lambda i: (0, i, 0)),
        ],
        out_shape=[
            jax.ShapeDtypeStruct((HEADS, N, 128), jnp.float32),
            jax.ShapeDtypeStruct((HEADS, N, 1), jnp.float32),
            jax.ShapeDtypeStruct((HEADS, N, 1), jnp.float32),
        ],
    )(x, W1, att_src1, att_dst1)


def _tc2_body(acc_ref, xph_ref, a1s_ref, a1d_ref, b1_ref, w2_ref, as2_ref,
              ad2_ref, xp2_ref, s2_ref, d2_ref, dinv_ref):
    parts = []
    for h in range(HEADS):
        a_self = a1s_ref[h, :, 0] + a1d_ref[h, :, 0]
        e_self = _lrelu_exp(a_self)
        num = acc_ref[h, :, :64] + e_self[:, None] * xph_ref[h, :, :64]
        den = acc_ref[h, :, 64] + e_self
        parts.append(_elu(num / den[:, None] + b1_ref[h][None, :]))
    h1 = jnp.concatenate(parts, axis=1)
    deg = acc_ref[0, :, 65] + 1.0
    dinv_ref[...] = lax.rsqrt(deg)[:, None]
    xp2 = jnp.dot(h1, w2_ref[...], preferred_element_type=jnp.float32)
    xp2_ref[...] = jnp.concatenate([xp2, jnp.zeros_like(xp2)], axis=1)
    s2_ref[...] = jnp.sum(xp2 * as2_ref[...], axis=1, keepdims=True)
    d2_ref[...] = jnp.sum(xp2 * ad2_ref[...], axis=1, keepdims=True)


def _tc2(accA, xph, a1s, a1d, b1r, W2, att_src2, att_dst2):
    return pl.pallas_call(
        _tc2_body,
        grid=(_GRID,),
        in_specs=[
            pl.BlockSpec((HEADS, _BN, 128), lambda i: (0, i, 0)),
            pl.BlockSpec((HEADS, _BN, 128), lambda i: (0, i, 0)),
            pl.BlockSpec((HEADS, _BN, 1), lambda i: (0, i, 0)),
            pl.BlockSpec((HEADS, _BN, 1), lambda i: (0, i, 0)),
            pl.BlockSpec((HEADS, HID), lambda i: (0, 0)),
            pl.BlockSpec((256, HID), lambda i: (0, 0)),
            pl.BlockSpec((1, HID), lambda i: (0, 0)),
            pl.BlockSpec((1, HID), lambda i: (0, 0)),
        ],
        out_specs=[
            pl.BlockSpec((_BN, 128), lambda i: (i, 0)),
            pl.BlockSpec((_BN, 1), lambda i: (i, 0)),
            pl.BlockSpec((_BN, 1), lambda i: (i, 0)),
            pl.BlockSpec((_BN, 1), lambda i: (i, 0)),
        ],
        out_shape=[
            jax.ShapeDtypeStruct((N, 128), jnp.float32),
            jax.ShapeDtypeStruct((N, 1), jnp.float32),
            jax.ShapeDtypeStruct((N, 1), jnp.float32),
            jax.ShapeDtypeStruct((N, 1), jnp.float32),
        ],
    )(accA, xph, a1s, a1d, b1r, W2, att_src2, att_dst2)


def _tc3_body(acc_ref, xp2_ref, s2_ref, d2_ref, dinv_ref, b2_ref, g_ref):
    a_self = s2_ref[:, 0] + d2_ref[:, 0]
    e_self = _lrelu_exp(a_self)
    num = (acc_ref[0, :, :64] + acc_ref[1, :, :64] +
           e_self[:, None] * xp2_ref[:, :64])
    den = acc_ref[0, :, 64] + acc_ref[1, :, 64] + e_self
    h2 = _elu(num / den[:, None] + b2_ref[...])
    g = h2 * dinv_ref[...]
    g_ref[...] = jnp.concatenate([g, jnp.zeros_like(g)], axis=1)


def _tc3(accB, xp2, s2, d2, dinv, b2r):
    return pl.pallas_call(
        _tc3_body,
        grid=(_GRID,),
        in_specs=[
            pl.BlockSpec((2, _BN, 128), lambda i: (0, i, 0)),
            pl.BlockSpec((_BN, 128), lambda i: (i, 0)),
            pl.BlockSpec((_BN, 1), lambda i: (i, 0)),
            pl.BlockSpec((_BN, 1), lambda i: (i, 0)),
            pl.BlockSpec((_BN, 1), lambda i: (i, 0)),
            pl.BlockSpec((1, HID), lambda i: (0, 0)),
        ],
        out_specs=[pl.BlockSpec((_BN, 128), lambda i: (i, 0))],
        out_shape=[jax.ShapeDtypeStruct((N, 128), jnp.float32)],
    )(accB, xp2, s2, d2, dinv, b2r)[0]


def _tc4_body(acc_ref, g_ref, dinv_ref, wm_ref, wl_ref, bm_ref, bl_ref,
              mu_ref, ls_ref):
    og = (acc_ref[0, :, :64] + acc_ref[1, :, :64] + g_ref[:, :64]) * dinv_ref[...]
    mu_ref[...] = jnp.dot(og, wm_ref[...],
                          preferred_element_type=jnp.float32) + bm_ref[...]
    ls_ref[...] = jnp.dot(og, wl_ref[...],
                          preferred_element_type=jnp.float32) + bl_ref[...]


def _tc4(accC, g, dinv, W_mu, W_ls, bmr, blr):
    return pl.pallas_call(
        _tc4_body,
        grid=(_GRID,),
        in_specs=[
            pl.BlockSpec((2, _BN, 128), lambda i: (0, i, 0)),
            pl.BlockSpec((_BN, 128), lambda i: (i, 0)),
            pl.BlockSpec((_BN, 1), lambda i: (i, 0)),
            pl.BlockSpec((HID, OUT_CH), lambda i: (0, 0)),
            pl.BlockSpec((HID, OUT_CH), lambda i: (0, 0)),
            pl.BlockSpec((1, OUT_CH), lambda i: (0, 0)),
            pl.BlockSpec((1, OUT_CH), lambda i: (0, 0)),
        ],
        out_specs=[
            pl.BlockSpec((_BN, OUT_CH), lambda i: (i, 0)),
            pl.BlockSpec((_BN, OUT_CH), lambda i: (i, 0)),
        ],
        out_shape=[
            jax.ShapeDtypeStruct((N, OUT_CH), jnp.float32),
            jax.ShapeDtypeStruct((N, OUT_CH), jnp.float32),
        ],
    )(accC, g, dinv, W_mu, W_ls, bmr, blr)


def _zero_buf(buf, nrows, ncols):
    zero16 = jnp.zeros((16,), jnp.float32)

    def zrow(r, _):
        for k in range(ncols // 16):
            buf[r, pl.ds(16 * k, 16)] = zero16
        return 0

    lax.fori_loop(0, nrows, zrow, 0)


@functools.partial(
    pl.kernel,
    out_type=jax.ShapeDtypeStruct((HEADS, _NPAD, 128), jnp.float32),
    mesh=_MESH,
    compiler_params=_SC_PARAMS,
    scratch_types=[
        _VSH((_NPAD, 128), jnp.float32),
        pltpu.VMEM((N,), jnp.float32),
        pltpu.VMEM((N,), jnp.float32),
        pltpu.VMEM((_B,), jnp.int32),
        pltpu.VMEM((_B,), jnp.int32),
        pltpu.VMEM((_B,), jnp.int32),
        pltpu.VMEM((_B, 128), jnp.float32),
        pltpu.VMEM((_B, 128), jnp.float32),
        pltpu.VMEM((_B,), jnp.float32),
        pltpu.SemaphoreType.DMA,
    ],
)
def _pass_a(xph_hbm, a1s_hbm, a1d_hbm, src_hbm, dst_hbm, out_hbm,
            acc, atbl_s, atbl_d, idx_s, idx_d, idx_g, gbuf, sbuf, ebuf,
            sem):
    c = lax.axis_index("c")
    s = lax.axis_index("s")

    row0 = s * _RPT
    iota = lax.iota(jnp.int32, 16)
    oh0 = jnp.where(iota == 0, 1.0, 0.0).astype(jnp.float32)
    oh1 = jnp.where(iota == 1, 1.0, 0.0).astype(jnp.float32)
    dflag = jnp.where(c == 0, 1.0, 0.0).astype(jnp.float32)
    ebase = s * _EPT_A

    for k in range(2):
        h = c + 2 * k
        _zero_buf(sbuf, _B, 128)
        for z in range(_RPT // _B):
            pltpu.sync_copy(sbuf, acc.at[pl.ds(row0 + z * _B, _B)])
        pltpu.sync_copy(a1s_hbm.at[h], atbl_s)
        pltpu.sync_copy(a1d_hbm.at[h], atbl_d)
        plsc.subcore_barrier()

        hoff = h * N
        tail_deg = oh1 * dflag if k == 0 else None

        def chunk(i, _):
            off = ebase + i * _B
            pltpu.sync_copy(src_hbm.at[pl.ds(off, _B)], idx_s)
            pltpu.sync_copy(dst_hbm.at[pl.ds(off, _B)], idx_d)

            def alphas(j, _):
                sv = idx_s[pl.ds(16 * j, 16)]
                dv = idx_d[pl.ds(16 * j, 16)]
                idx_g[pl.ds(16 * j, 16)] = sv + hoff
                a = (plsc.load_gather(atbl_s, [sv]) +
                     plsc.load_gather(atbl_d, [dv]))
                ebuf[pl.ds(16 * j, 16)] = _lrelu_exp(a)
                return 0

            lax.fori_loop(0, _B // 16, alphas, 0)
            pltpu.async_copy(xph_hbm.at[idx_g], gbuf, sem).wait()

            def scale(j, _):
                ev = ebuf[pl.ds(16 * j, 16)]
                for t in range(16):
                    r = 16 * j + t
                    e0 = ev[t]
                    for q in range(4):
                        sbuf[r, pl.ds(16 * q, 16)] = (
                            gbuf[r, pl.ds(16 * q, 16)] * e0)
                    if tail_deg is not None:
                        sbuf[r, pl.ds(64, 16)] = e0 * oh0 + tail_deg
                    else:
                        sbuf[r, pl.ds(64, 16)] = e0 * oh0
                return 0

            lax.fori_loop(0, _B // 16, scale, 0)
            pltpu.sync_copy(sbuf, acc.at[idx_d], add=True)
            return 0

        lax.fori_loop(0, _EPT_A // _B, chunk, 0)
        plsc.subcore_barrier()
        pltpu.sync_copy(acc.at[pl.ds(row0, _RPT)],
                        out_hbm.at[h, pl.ds(row0, _RPT)])


@functools.partial(
    pl.kernel,
    out_type=jax.ShapeDtypeStruct((2, _NPAD, 128), jnp.float32),
    mesh=_MESH,
    compiler_params=_SC_PARAMS,
    scratch_types=[
        _VSH((_NPAD, 128), jnp.float32),
        pltpu.VMEM((N,), jnp.float32),
        pltpu.VMEM((N,), jnp.float32),
        pltpu.VMEM((_B,), jnp.int32),
        pltpu.VMEM((_B,), jnp.int32),
        pltpu.VMEM((_B, 128), jnp.float32),
        pltpu.VMEM((_B, 128), jnp.float32),
        pltpu.VMEM((_B,), jnp.float32),
        pltpu.SemaphoreType.DMA,
    ],
)
def _pass_b(xp2_hbm, s2_hbm, d2_hbm, src_hbm, dst_hbm, out_hbm,
            acc, atbl_s, atbl_d, idx_s, idx_d, gbuf, sbuf, ebuf, sem):
    c = lax.axis_index("c")
    s = lax.axis_index("s")

    _zero_buf(sbuf, _B, 128)
    row0 = s * _RPT
    for z in range(_RPT // _B):
        pltpu.sync_copy(sbuf, acc.at[pl.ds(row0 + z * _B, _B)])
    pltpu.sync_copy(s2_hbm, atbl_s)
    pltpu.sync_copy(d2_hbm, atbl_d)
    plsc.subcore_barrier()

    iota = lax.iota(jnp.int32, 16)
    oh0 = jnp.where(iota == 0, 1.0, 0.0).astype(jnp.float32)
    ebase = (s * 2 + c) * _EPT_BC

    def chunk(i, _):
        off = ebase + i * _B
        pltpu.sync_copy(src_hbm.at[pl.ds(off, _B)], idx_s)
        pltpu.sync_copy(dst_hbm.at[pl.ds(off, _B)], idx_d)

        def alphas(j, _):
            sv = idx_s[pl.ds(16 * j, 16)]
            dv = idx_d[pl.ds(16 * j, 16)]
            a = (plsc.load_gather(atbl_s, [sv]) +
                 plsc.load_gather(atbl_d, [dv]))
            ebuf[pl.ds(16 * j, 16)] = _lrelu_exp(a)
            return 0

        lax.fori_loop(0, _B // 16, alphas, 0)
        pltpu.async_copy(xp2_hbm.at[idx_s], gbuf, sem).wait()

        def scale(j, _):
            ev = ebuf[pl.ds(16 * j, 16)]
            for t in range(16):
                r = 16 * j + t
                e0 = ev[t]
                for q in range(4):
                    sbuf[r, pl.ds(16 * q, 16)] = gbuf[r, pl.ds(16 * q, 16)] * e0
                sbuf[r, pl.ds(64, 16)] = e0 * oh0
            return 0

        lax.fori_loop(0, _B // 16, scale, 0)
        pltpu.sync_copy(sbuf, acc.at[idx_d], add=True)
        return 0

    lax.fori_loop(0, _EPT_BC // _B, chunk, 0)
    plsc.subcore_barrier()
    pltpu.sync_copy(acc.at[pl.ds(row0, _RPT)],
                    out_hbm.at[c, pl.ds(row0, _RPT)])


@functools.partial(
    pl.kernel,
    out_type=jax.ShapeDtypeStruct((2, _NPAD, 128), jnp.float32),
    mesh=_MESH,
    compiler_params=_SC_PARAMS,
    scratch_types=[
        _VSH((_NPAD, 128), jnp.float32),
        pltpu.VMEM((_B,), jnp.int32),
        pltpu.VMEM((_B,), jnp.int32),
        pltpu.VMEM((_B, 128), jnp.float32),
        pltpu.SemaphoreType.DMA,
    ],
)
def _pass_c(g_hbm, src_hbm, dst_hbm, out_hbm,
            acc, idx_s, idx_d, gbuf, sem):
    c = lax.axis_index("c")
    s = lax.axis_index("s")

    _zero_buf(gbuf, _B, 128)
    row0 = s * _RPT
    for z in range(_RPT // _B):
        pltpu.sync_copy(gbuf, acc.at[pl.ds(row0 + z * _B, _B)])
    plsc.subcore_barrier()

    ebase = (s * 2 + c) * _EPT_BC

    def chunk(i, _):
        off = ebase + i * _B
        pltpu.sync_copy(src_hbm.at[pl.ds(off, _B)], idx_s)
        pltpu.sync_copy(dst_hbm.at[pl.ds(off, _B)], idx_d)
        pltpu.async_copy(g_hbm.at[idx_s], gbuf, sem).wait()
        pltpu.sync_copy(gbuf, acc.at[idx_d], add=True)
        return 0

    lax.fori_loop(0, _EPT_BC // _B, chunk, 0)
    plsc.subcore_barrier()
    pltpu.sync_copy(acc.at[pl.ds(row0, _RPT)],
                    out_hbm.at[c, pl.ds(row0, _RPT)])


def kernel(x, edge_index, W1, att_src1, att_dst1, b1, W2, att_src2, att_dst2,
           b2, W_mu, b_mu, W_ls, b_ls):
    src = edge_index[0]
    dst = edge_index[1]

    xph, a1s, a1d = _tc1(x, W1, att_src1, att_dst1)
    accA = _pass_a(xph.reshape(HEADS * N, 128), a1s.reshape(HEADS, N),
                   a1d.reshape(HEADS, N), src, dst)
    xp2, s2, d2, dinv = _tc2(accA, xph, a1s, a1d, b1.reshape(HEADS, HID),
                             W2, att_src2, att_dst2)
    accB = _pass_b(xp2, s2.reshape(N), d2.reshape(N), src, dst)
    g = _tc3(accB, xp2, s2, d2, dinv, b2.reshape(1, HID))
    accC = _pass_c(g, src, dst)
    mu, ls = _tc4(accC, g, dinv, W_mu, W_ls, b_mu.reshape(1, OUT_CH),
                  b_ls.reshape(1, OUT_CH))
    return (mu, ls)


# speedup vs baseline: 44.1473x; 1.7055x over previous
import functools

import jax
import jax.numpy as jnp
from jax import lax
from jax.experimental import pallas as pl
from jax.experimental.pallas import tpu as pltpu
from jax.experimental.pallas import tpu_sc as plsc

N = 10000
E = 320000
IN_CH = 128
HID = 64
OUT_CH = 32
HEADS = 4

_BN = 400
_GRID = N // _BN

_B = 80
_NT = 16
_NPAD = 10240
_RPT = _NPAD // _NT
_EPT_A = E // _NT
_EPT_BC = E // (2 * _NT)

_MESH = plsc.VectorSubcoreMesh(core_axis_name="c", subcore_axis_name="s")
_VSH = pltpu.MemorySpace.VMEM_SHARED
_SC_PARAMS = pltpu.CompilerParams(needs_layout_passes=False)


def _lrelu_exp(a):
    return jnp.exp(jnp.maximum(a, 0.2 * a))


def _elu(z):
    return jnp.where(z > 0, z, jnp.exp(z) - 1.0)


def _tc1_body(x_ref, w1_ref, as_ref, ad_ref, xph_ref, a1s_ref, a1d_ref):
    xp = jnp.dot(x_ref[...], w1_ref[...], preferred_element_type=jnp.float32)
    for h in range(HEADS):
        seg = xp[:, h * 64:(h + 1) * 64]
        xph_ref[h] = jnp.concatenate([seg, jnp.zeros_like(seg)], axis=1)
        a1s_ref[h, :, 0] = jnp.sum(seg * as_ref[h][None, :], axis=1)
        a1d_ref[h, :, 0] = jnp.sum(seg * ad_ref[h][None, :], axis=1)


def _tc1(x, W1, att_src1, att_dst1):
    return pl.pallas_call(
        _tc1_body,
        grid=(_GRID,),
        in_specs=[
            pl.BlockSpec((_BN, IN_CH), lambda i: (i, 0)),
            pl.BlockSpec((IN_CH, 256), lambda i: (0, 0)),
            pl.BlockSpec((HEADS, HID), lambda i: (0, 0)),
            pl.BlockSpec((HEADS, HID), lambda i: (0, 0)),
        ],
        out_specs=[
            pl.BlockSpec((HEADS, _BN, 128), lambda i: (0, i, 0)),
            pl.BlockSpec((HEADS, _BN, 1), lambda i: (0, i, 0)),
            pl.BlockSpec((HEADS, _BN, 1), lambda i: (0, i, 0)),
        ],
        out_shape=[
            jax.ShapeDtypeStruct((HEADS, N, 128), jnp.float32),
            jax.ShapeDtypeStruct((HEADS, N, 1), jnp.float32),
            jax.ShapeDtypeStruct((HEADS, N, 1), jnp.float32),
        ],
    )(x, W1, att_src1, att_dst1)


def _tc2_body(acc_ref, xph_ref, a1s_ref, a1d_ref, b1_ref, w2_ref, as2_ref,
              ad2_ref, xp2_ref, s2_ref, d2_ref, dinv_ref):
    parts = []
    for h in range(HEADS):
        a_self = a1s_ref[h, :, 0] + a1d_ref[h, :, 0]
        e_self = _lrelu_exp(a_self)
        num = acc_ref[h, :, :64] + e_self[:, None] * xph_ref[h, :, :64]
        den = acc_ref[h, :, 64] + e_self
        parts.append(_elu(num / den[:, None] + b1_ref[h][None, :]))
    h1 = jnp.concatenate(parts, axis=1)
    deg = acc_ref[0, :, 65] + 1.0
    dinv_ref[...] = lax.rsqrt(deg)[:, None]
    xp2 = jnp.dot(h1, w2_ref[...], preferred_element_type=jnp.float32)
    xp2_ref[...] = jnp.concatenate([xp2, jnp.zeros_like(xp2)], axis=1)
    s2_ref[...] = jnp.sum(xp2 * as2_ref[...], axis=1, keepdims=True)
    d2_ref[...] = jnp.sum(xp2 * ad2_ref[...], axis=1, keepdims=True)


def _tc2(accA, xph, a1s, a1d, b1r, W2, att_src2, att_dst2):
    return pl.pallas_call(
        _tc2_body,
        grid=(_GRID,),
        in_specs=[
            pl.BlockSpec((HEADS, _BN, 128), lambda i: (0, i, 0)),
            pl.BlockSpec((HEADS, _BN, 128), lambda i: (0, i, 0)),
            pl.BlockSpec((HEADS, _BN, 1), lambda i: (0, i, 0)),
            pl.BlockSpec((HEADS, _BN, 1), lambda i: (0, i, 0)),
            pl.BlockSpec((HEADS, HID), lambda i: (0, 0)),
            pl.BlockSpec((256, HID), lambda i: (0, 0)),
            pl.BlockSpec((1, HID), lambda i: (0, 0)),
            pl.BlockSpec((1, HID), lambda i: (0, 0)),
        ],
        out_specs=[
            pl.BlockSpec((_BN, 128), lambda i: (i, 0)),
            pl.BlockSpec((_BN, 1), lambda i: (i, 0)),
            pl.BlockSpec((_BN, 1), lambda i: (i, 0)),
            pl.BlockSpec((_BN, 1), lambda i: (i, 0)),
        ],
        out_shape=[
            jax.ShapeDtypeStruct((N, 128), jnp.float32),
            jax.ShapeDtypeStruct((N, 1), jnp.float32),
            jax.ShapeDtypeStruct((N, 1), jnp.float32),
            jax.ShapeDtypeStruct((N, 1), jnp.float32),
        ],
    )(accA, xph, a1s, a1d, b1r, W2, att_src2, att_dst2)


def _tc3_body(acc_ref, xp2_ref, s2_ref, d2_ref, dinv_ref, b2_ref, g_ref):
    a_self = s2_ref[:, 0] + d2_ref[:, 0]
    e_self = _lrelu_exp(a_self)
    num = (acc_ref[0, :, :64] + acc_ref[1, :, :64] +
           e_self[:, None] * xp2_ref[:, :64])
    den = acc_ref[0, :, 64] + acc_ref[1, :, 64] + e_self
    h2 = _elu(num / den[:, None] + b2_ref[...])
    g = h2 * dinv_ref[...]
    g_ref[...] = jnp.concatenate([g, jnp.zeros_like(g)], axis=1)


def _tc3(accB, xp2, s2, d2, dinv, b2r):
    return pl.pallas_call(
        _tc3_body,
        grid=(_GRID,),
        in_specs=[
            pl.BlockSpec((2, _BN, 128), lambda i: (0, i, 0)),
            pl.BlockSpec((_BN, 128), lambda i: (i, 0)),
            pl.BlockSpec((_BN, 1), lambda i: (i, 0)),
            pl.BlockSpec((_BN, 1), lambda i: (i, 0)),
            pl.BlockSpec((_BN, 1), lambda i: (i, 0)),
            pl.BlockSpec((1, HID), lambda i: (0, 0)),
        ],
        out_specs=[pl.BlockSpec((_BN, 128), lambda i: (i, 0))],
        out_shape=[jax.ShapeDtypeStruct((N, 128), jnp.float32)],
    )(accB, xp2, s2, d2, dinv, b2r)[0]


def _tc4_body(acc_ref, g_ref, dinv_ref, wm_ref, wl_ref, bm_ref, bl_ref,
              mu_ref, ls_ref):
    og = (acc_ref[0, :, :64] + acc_ref[1, :, :64] + g_ref[:, :64]) * dinv_ref[...]
    mu_ref[...] = jnp.dot(og, wm_ref[...],
                          preferred_element_type=jnp.float32) + bm_ref[...]
    ls_ref[...] = jnp.dot(og, wl_ref[...],
                          preferred_element_type=jnp.float32) + bl_ref[...]


def _tc4(accC, g, dinv, W_mu, W_ls, bmr, blr):
    return pl.pallas_call(
        _tc4_body,
        grid=(_GRID,),
        in_specs=[
            pl.BlockSpec((2, _BN, 128), lambda i: (0, i, 0)),
            pl.BlockSpec((_BN, 128), lambda i: (i, 0)),
            pl.BlockSpec((_BN, 1), lambda i: (i, 0)),
            pl.BlockSpec((HID, OUT_CH), lambda i: (0, 0)),
            pl.BlockSpec((HID, OUT_CH), lambda i: (0, 0)),
            pl.BlockSpec((1, OUT_CH), lambda i: (0, 0)),
            pl.BlockSpec((1, OUT_CH), lambda i: (0, 0)),
        ],
        out_specs=[
            pl.BlockSpec((_BN, OUT_CH), lambda i: (i, 0)),
            pl.BlockSpec((_BN, OUT_CH), lambda i: (i, 0)),
        ],
        out_shape=[
            jax.ShapeDtypeStruct((N, OUT_CH), jnp.float32),
            jax.ShapeDtypeStruct((N, OUT_CH), jnp.float32),
        ],
    )(accC, g, dinv, W_mu, W_ls, bmr, blr)


def _zero_buf(buf, nrows, ncols):
    zero16 = jnp.zeros((16,), jnp.float32)

    def zrow(r, _):
        for k in range(ncols // 16):
            buf[r, pl.ds(16 * k, 16)] = zero16
        return 0

    lax.fori_loop(0, nrows, zrow, 0)


def _zero_acc(acc, stage, row0):
    _zero_buf(stage, _B, 128)
    for z in range(_RPT // _B):
        pltpu.sync_copy(stage, acc.at[pl.ds(row0 + z * _B, _B)])


def _scale_rows(gbuf, ebuf, tail_vecs):
    def scale(j, _):
        ev = ebuf[pl.ds(16 * j, 16)]
        for t in range(16):
            r = 16 * j + t
            e0 = ev[t]
            for q in range(4):
                gbuf[r, pl.ds(16 * q, 16)] = gbuf[r, pl.ds(16 * q, 16)] * e0
            tail = e0 * tail_vecs[0]
            if tail_vecs[1] is not None:
                tail = tail + tail_vecs[1]
            gbuf[r, pl.ds(64, 16)] = tail
        return 0

    lax.fori_loop(0, _B // 16, scale, 0)


def _edge_pipeline(ei_hbm, tbl_hbm, acc, bufs, sems, cbase, n_c, prep, scale):
    semi = sems[0:2]
    semg = sems[2:4]
    semsc = sems[4]

    def eslice(i):
        return ei_hbm.at[cbase + i]

    def gref(b):
        idxb, idxg, _, _, _ = b
        return idxg if idxg is not None else idxb.at[0]

    def gather_start(b, p):
        pltpu.async_copy(tbl_hbm.at[gref(b)], b[4], semg[p])
        return None

    def gather_wait(b, p):
        pltpu.make_async_copy(tbl_hbm.at[gref(b)], b[4], semg[p]).wait()

    def scat_start(b):
        pltpu.async_copy(b[4], acc.at[b[2]], semsc, add=True)

    def scat_wait(b):
        pltpu.make_async_copy(b[4], acc.at[b[2]], semsc).wait()

    def step(i, cur, nxt, p):
        gather_wait(cur, p)
        pl.when(i > 0)(lambda: scat_wait(nxt))
        if scale is not None:
            scale(cur)
        scat_start(cur)

        def prefetch():
            pltpu.make_async_copy(eslice(i + 1), nxt[0], semi[1 - p]).wait()
            prep(nxt)
            gather_start(nxt, 1 - p)
            return None

        pl.when(i + 1 < n_c)(prefetch)
        def idx_prefetch():
            pltpu.async_copy(eslice(i + 2), cur[0], semi[p])
            return None

        pl.when(i + 2 < n_c)(idx_prefetch)

    pltpu.sync_copy(eslice(0), bufs[0][0])
    prep(bufs[0])
    gather_start(bufs[0], 0)
    if n_c > 1:
        pltpu.async_copy(eslice(1), bufs[1][0], semi[1])

    def double_step(m, _):
        step(2 * m, bufs[0], bufs[1], 0)
        step(2 * m + 1, bufs[1], bufs[0], 1)
        return 0

    lax.fori_loop(0, n_c // 2, double_step, 0)
    if n_c % 2:
        p = (n_c - 1) % 2
        step(n_c - 1, bufs[p], bufs[1 - p], p)
    scat_wait(bufs[(n_c - 1) % 2])


def _sc_scratch(with_tables, with_alpha):
    sc = [_VSH((_NPAD, 128), jnp.float32)]
    if with_tables:
        sc += [pltpu.VMEM((N,), jnp.float32), pltpu.VMEM((N,), jnp.float32)]
    for _ in range(2):
        sc.append(pltpu.VMEM((2, _B), jnp.int32))
        if with_alpha:
            sc.append(pltpu.VMEM((_B,), jnp.int32))
        sc.append(pltpu.VMEM((_B,), jnp.int32))
        if with_alpha:
            sc.append(pltpu.VMEM((_B,), jnp.float32))
        sc.append(pltpu.VMEM((_B, 128), jnp.float32))
    sc += [pltpu.SemaphoreType.DMA] * 5
    return sc


@functools.partial(
    pl.kernel,
    out_type=jax.ShapeDtypeStruct((HEADS, _NPAD, 128), jnp.float32),
    mesh=_MESH,
    compiler_params=_SC_PARAMS,
    scratch_types=_sc_scratch(True, True),
)
def _pass_a(xph_hbm, a1s_hbm, a1d_hbm, ei_hbm, out_hbm,
            acc, atbl_s, atbl_d,
            idxb0, idxg0, idxsc0, ebuf0, gbuf0,
            idxb1, idxg1, idxsc1, ebuf1, gbuf1,
            semi0, semi1, semg0, semg1, semsc):
    c = lax.axis_index("c")
    s = lax.axis_index("s")
    row0 = s * _RPT
    iota = lax.iota(jnp.int32, 16)
    oh0 = jnp.where(iota == 0, 1.0, 0.0).astype(jnp.float32)
    oh1 = jnp.where(iota == 1, 1.0, 0.0).astype(jnp.float32)
    dflag = jnp.where(c == 0, 1.0, 0.0).astype(jnp.float32)
    cbase = s * (_EPT_A // _B)
    bufs = ((idxb0, idxg0, idxsc0, ebuf0, gbuf0),
            (idxb1, idxg1, idxsc1, ebuf1, gbuf1))
    sems = (semi0, semi1, semg0, semg1, semsc)

    for k in range(2):
        h = c + 2 * k
        _zero_acc(acc, gbuf0, row0)
        pltpu.sync_copy(a1s_hbm.at[h], atbl_s)
        pltpu.sync_copy(a1d_hbm.at[h], atbl_d)
        plsc.subcore_barrier()

        hoff = h * N
        tail_vecs = (oh0, oh1 * dflag if k == 0 else None)

        def prep(b):
            idxb, idxg, idxsc, ebuf, _ = b
            for j in range(_B // 16):
                ds = pl.ds(16 * j, 16)
                sv = idxb[0, ds]
                dv = idxb[1, ds]
                idxg[ds] = sv + hoff
                idxsc[ds] = dv
                a = (plsc.load_gather(atbl_s, [sv]) +
                     plsc.load_gather(atbl_d, [dv]))
                ebuf[ds] = _lrelu_exp(a)

        def scale(b):
            _scale_rows(b[4], b[3], tail_vecs)

        _edge_pipeline(ei_hbm, xph_hbm, acc, bufs, sems, cbase,
                       _EPT_A // _B, prep, scale)
        plsc.subcore_barrier()
        pltpu.sync_copy(acc.at[pl.ds(row0, _RPT)],
                        out_hbm.at[h, pl.ds(row0, _RPT)])


@functools.partial(
    pl.kernel,
    out_type=jax.ShapeDtypeStruct((2, _NPAD, 128), jnp.float32),
    mesh=_MESH,
    compiler_params=_SC_PARAMS,
    scratch_types=_sc_scratch(True, True),
)
def _pass_b(xp2_hbm, s2_hbm, d2_hbm, ei_hbm, out_hbm,
            acc, atbl_s, atbl_d,
            idxb0, idxg0, idxsc0, ebuf0, gbuf0,
            idxb1, idxg1, idxsc1, ebuf1, gbuf1,
            semi0, semi1, semg0, semg1, semsc):
    c = lax.axis_index("c")
    s = lax.axis_index("s")
    row0 = s * _RPT
    _zero_acc(acc, gbuf0, row0)
    pltpu.sync_copy(s2_hbm, atbl_s)
    pltpu.sync_copy(d2_hbm, atbl_d)
    plsc.subcore_barrier()

    iota = lax.iota(jnp.int32, 16)
    oh0 = jnp.where(iota == 0, 1.0, 0.0).astype(jnp.float32)
    cbase = (s * 2 + c) * (_EPT_BC // _B)
    bufs = ((idxb0, idxg0, idxsc0, ebuf0, gbuf0),
            (idxb1, idxg1, idxsc1, ebuf1, gbuf1))
    sems = (semi0, semi1, semg0, semg1, semsc)

    def prep(b):
        idxb, idxg, idxsc, ebuf, _ = b
        for j in range(_B // 16):
            ds = pl.ds(16 * j, 16)
            sv = idxb[0, ds]
            dv = idxb[1, ds]
            idxg[ds] = sv
            idxsc[ds] = dv
            a = (plsc.load_gather(atbl_s, [sv]) +
                 plsc.load_gather(atbl_d, [dv]))
            ebuf[ds] = _lrelu_exp(a)

    def scale(b):
        _scale_rows(b[4], b[3], (oh0, None))

    _edge_pipeline(ei_hbm, xp2_hbm, acc, bufs, sems, cbase,
                   _EPT_BC // _B, prep, scale)
    plsc.subcore_barrier()
    pltpu.sync_copy(acc.at[pl.ds(row0, _RPT)],
                    out_hbm.at[c, pl.ds(row0, _RPT)])


@functools.partial(
    pl.kernel,
    out_type=jax.ShapeDtypeStruct((2, _NPAD, 128), jnp.float32),
    mesh=_MESH,
    compiler_params=_SC_PARAMS,
    scratch_types=_sc_scratch(False, False),
)
def _pass_c(g_hbm, ei_hbm, out_hbm,
            acc,
            idxb0, idxsc0, gbuf0,
            idxb1, idxsc1, gbuf1,
            semi0, semi1, semg0, semg1, semsc):
    c = lax.axis_index("c")
    s = lax.axis_index("s")
    row0 = s * _RPT
    _zero_acc(acc, gbuf0, row0)
    plsc.subcore_barrier()

    cbase = (s * 2 + c) * (_EPT_BC // _B)
    bufs = ((idxb0, None, idxsc0, None, gbuf0),
            (idxb1, None, idxsc1, None, gbuf1))
    sems = (semi0, semi1, semg0, semg1, semsc)

    def prep(b):
        idxb, _, idxsc, _, _ = b
        for j in range(_B // 16):
            ds = pl.ds(16 * j, 16)
            idxsc[ds] = idxb[1, ds]

    _edge_pipeline(ei_hbm, g_hbm, acc, bufs, sems, cbase,
                   _EPT_BC // _B, prep, None)
    plsc.subcore_barrier()
    pltpu.sync_copy(acc.at[pl.ds(row0, _RPT)],
                    out_hbm.at[c, pl.ds(row0, _RPT)])


def kernel(x, edge_index, W1, att_src1, att_dst1, b1, W2, att_src2, att_dst2,
           b2, W_mu, b_mu, W_ls, b_ls):
    ei3 = edge_index.reshape(2, E // _B, _B).transpose(1, 0, 2)
    xph, a1s, a1d = _tc1(x, W1, att_src1, att_dst1)
    accA = _pass_a(xph.reshape(HEADS * N, 128), a1s.reshape(HEADS, N),
                   a1d.reshape(HEADS, N), ei3)
    xp2, s2, d2, dinv = _tc2(accA, xph, a1s, a1d, b1.reshape(HEADS, HID),
                             W2, att_src2, att_dst2)
    accB = _pass_b(xp2, s2.reshape(N), d2.reshape(N), ei3)
    g = _tc3(accB, xp2, s2, d2, dinv, b2.reshape(1, HID))
    accC = _pass_c(g, ei3)
    mu, ls = _tc4(accC, g, dinv, W_mu, W_ls, b_mu.reshape(1, OUT_CH),
                  b_ls.reshape(1, OUT_CH))
    return (mu, ls)
